# Initial kernel scaffold; baseline (speedup 1.0000x reference)
import jax
import jax.numpy as jnp
from jax.experimental import pallas as pl


def kernel(data, org_edge_index, v1, g1, b1, v2, g2, b2, v3, g3, b3, emb1, emb2):
    raise NotImplementedError("write your pallas kernel here")


# speedup vs baseline: 1.2196x; 1.2196x over previous
import functools

import jax
import jax.numpy as jnp
from jax import lax
from jax.experimental import pallas as pl
from jax.experimental.pallas import tpu as pltpu

NODE_NUM = 100
BATCH = 1024
FEAT = 128
SENSOR_DIM = 64
TOPK = 20
KSIZE = 3
N_EDGES = 2000

BBLK = 16
EBLK = 128


def _shift_time(h, s):
    if s == 0:
        return h
    bblk, l, c = h.shape
    z = jnp.zeros((bblk, s, c), dtype=h.dtype)
    return jnp.concatenate([z, h[:, : l - s, :]], axis=1)


def _tcn_kernel(x_ref, v1_ref, g1_ref, b1_ref, v2_ref, g2_ref, b2_ref,
                v3_ref, g3_ref, b3_ref, o_ref):
    x = x_ref[...]
    xt = jnp.transpose(x, (0, 2, 1))
    h = xt
    vrefs = (v1_ref, v2_ref, v3_ref)
    grefs = (g1_ref, g2_ref, g3_ref)
    brefs = (b1_ref, b2_ref, b3_ref)
    for i, d in enumerate((1, 2, 4)):
        v = vrefs[i][...]
        g = grefs[i][...]
        b = brefs[i][...]
        sumsq = jnp.sum(v * v, axis=0)
        sumsq = jnp.sum(sumsq, axis=1, keepdims=True)
        scale = jnp.transpose(g, (1, 0)) / jnp.sqrt(sumsq + 1e-12)
        acc = b[None]
        for k in range(KSIZE):
            w = v[k] * scale
            hs = _shift_time(h, (KSIZE - 1 - k) * d)
            acc = acc + lax.dot_general(
                hs, w, (((2,), (1,)), ((), ())),
                preferred_element_type=jnp.float32)
        h = jnp.maximum(acc, 0.0)
    out = jnp.maximum(h + xt, 0.0)
    o_ref[...] = jnp.transpose(out, (0, 2, 1))


def _graph_kernel(e1_ref, e2_ref, masked_ref, idx_ref):
    w1 = e1_ref[...]
    w2 = e2_ref[...]
    dots = lax.dot_general(w1, w2, (((1,), (1,)), ((), ())),
                           preferred_element_type=jnp.float32)
    n1 = jnp.sqrt(jnp.sum(w1 * w1, axis=1, keepdims=True))
    n2 = jnp.sqrt(jnp.sum(w2 * w2, axis=1, keepdims=True))
    n2row = jnp.transpose(n2, (1, 0))
    cos = jnp.maximum(dots, 0.0) / (n1 * n2row)

    lane = lax.broadcasted_iota(jnp.int32, (NODE_NUM, NODE_NUM), 1)
    work = cos
    sel = jnp.zeros((NODE_NUM, NODE_NUM), jnp.bool_)
    idx_cols = []
    big = jnp.int32(NODE_NUM + 1)
    for _ in range(TOPK):
        m = jnp.max(work, axis=1, keepdims=True)
        cand = jnp.where(work == m, lane, big)
        chosen = jnp.min(cand, axis=1, keepdims=True)
        hit = lane == chosen
        sel = jnp.logical_or(sel, hit)
        work = jnp.where(hit, -1.0, work)
        idx_cols.append(chosen)
    masked_ref[...] = jnp.where(sel, cos, 0.0)
    idx_ref[...] = jnp.concatenate(idx_cols, axis=1)


def _edge_kernel(org_ref, o_ref):
    org = org_ref[...]
    i = pl.program_id(0)
    boff = lax.broadcasted_iota(jnp.int32, (1, EBLK, 1), 1)
    boff = (boff + i * EBLK) * NODE_NUM
    o_ref[...] = org + boff


def kernel(data, org_edge_index, v1, g1, b1, v2, g2, b2, v3, g3, b3,
           emb1, emb2):
    vt1 = jnp.transpose(v1, (2, 0, 1))
    vt2 = jnp.transpose(v2, (2, 0, 1))
    vt3 = jnp.transpose(v3, (2, 0, 1))
    g1r, b1r = g1.reshape(1, NODE_NUM), b1.reshape(1, NODE_NUM)
    g2r, b2r = g2.reshape(1, NODE_NUM), b2.reshape(1, NODE_NUM)
    g3r, b3r = g3.reshape(1, NODE_NUM), b3.reshape(1, NODE_NUM)

    full = lambda shape: pl.BlockSpec(shape, lambda i: (0,) * len(shape))
    x3 = pl.pallas_call(
        _tcn_kernel,
        grid=(BATCH // BBLK,),
        in_specs=[
            pl.BlockSpec((BBLK, NODE_NUM, FEAT), lambda i: (i, 0, 0)),
            full((KSIZE, NODE_NUM, NODE_NUM)), full((1, NODE_NUM)), full((1, NODE_NUM)),
            full((KSIZE, NODE_NUM, NODE_NUM)), full((1, NODE_NUM)), full((1, NODE_NUM)),
            full((KSIZE, NODE_NUM, NODE_NUM)), full((1, NODE_NUM)), full((1, NODE_NUM)),
        ],
        out_specs=pl.BlockSpec((BBLK, NODE_NUM, FEAT), lambda i: (i, 0, 0)),
        out_shape=jax.ShapeDtypeStruct((BATCH, NODE_NUM, FEAT), jnp.float32),
        compiler_params=pltpu.CompilerParams(
            dimension_semantics=("arbitrary",)),
    )(data, vt1, g1r, b1r, vt2, g2r, b2r, vt3, g3r, b3r)
    x = x3.reshape(-1, FEAT)

    masked, idx = pl.pallas_call(
        _graph_kernel,
        in_specs=[
            pl.BlockSpec((NODE_NUM, SENSOR_DIM), lambda: (0, 0)),
            pl.BlockSpec((NODE_NUM, SENSOR_DIM), lambda: (0, 0)),
        ],
        out_specs=[
            pl.BlockSpec((NODE_NUM, NODE_NUM), lambda: (0, 0)),
            pl.BlockSpec((NODE_NUM, TOPK), lambda: (0, 0)),
        ],
        out_shape=[
            jax.ShapeDtypeStruct((NODE_NUM, NODE_NUM), jnp.float32),
            jax.ShapeDtypeStruct((NODE_NUM, TOPK), jnp.int32),
        ],
    )(emb1, emb2)

    org3 = org_edge_index.reshape(2, 1, N_EDGES)
    edges = pl.pallas_call(
        _edge_kernel,
        grid=(BATCH // EBLK,),
        in_specs=[pl.BlockSpec((2, 1, N_EDGES), lambda i: (0, 0, 0))],
        out_specs=pl.BlockSpec((2, EBLK, N_EDGES), lambda i: (0, i, 0)),
        out_shape=jax.ShapeDtypeStruct((2, BATCH, N_EDGES), jnp.int32),
        compiler_params=pltpu.CompilerParams(
            dimension_semantics=("arbitrary",)),
    )(org3)
    batch_edge_index = edges.reshape(2, BATCH * N_EDGES)

    return x, masked, idx, batch_edge_index
